# baseline (speedup 1.0000x reference)
import functools

import jax
import jax.numpy as jnp
from jax import lax
from jax.experimental import pallas as pl
from jax.experimental.pallas import tpu as pltpu
from jax.experimental.pallas import tpu_sc as plsc

N = 8192
D = 1024
E = 8
LANES = 128
BN = 512
T = 256
NT = N // T + E
PADDED_N = NT * T

NW = 32
RPW = N // NW
C = 32

f32 = jnp.float32
i32 = jnp.int32


def _route_body(x_ref, wg_ref, bg_ref, slot_ref, te_ref, idx_s, rank_s, cnt_s):
    p = pl.program_id(0)
    i = pl.program_id(1)

    @pl.when(p == 0)
    def _phase0():
        logits = jnp.dot(x_ref[...], wg_ref[...],
                         preferred_element_type=f32) + bg_ref[...]
        m = jnp.max(logits, axis=1, keepdims=True)
        lane = lax.broadcasted_iota(i32, logits.shape, 1)
        idxv = jnp.min(jnp.where(logits == m, lane, LANES - 1), axis=1,
                       keepdims=True)
        oh = (lane == idxv).astype(f32)
        r = lax.broadcasted_iota(i32, (BN, BN), 0)
        c = lax.broadcasted_iota(i32, (BN, BN), 1)
        ltri = (r > c).astype(f32)
        csum = jnp.dot(ltri, oh, preferred_element_type=f32)
        carry = jnp.where(i == 0, jnp.zeros((1, LANES), f32), cnt_s[...])
        rank = jnp.sum(oh * (csum + carry), axis=1, keepdims=True)
        idx_s[pl.ds(i * BN, BN), :] = idxv
        rank_s[pl.ds(i * BN, BN), :] = rank.astype(i32)
        cnt_s[...] = carry + jnp.sum(oh, axis=0, keepdims=True)

    @pl.when(p == 1)
    def _phase1():
        starts, ends = [], []
        cum = i32(0)
        for e in range(E):
            cnt = cnt_s[0, e].astype(i32)
            padded = ((cnt + (T - 1)) // T) * T
            starts.append(cum)
            cum = cum + padded
            ends.append(cum)
        idxb = idx_s[pl.ds(i * BN, BN), :]
        slot = rank_s[pl.ds(i * BN, BN), :]
        for e in range(E):
            slot = slot + jnp.where(idxb == e, starts[e], 0)
        slot_ref[...] = slot

        @pl.when(i == 0)
        def _():
            t = lax.broadcasted_iota(i32, (1, LANES), 1) * T
            acc = jnp.zeros((1, LANES), i32)
            for e in range(E):
                acc = acc + (t >= ends[e]).astype(i32)
            te_ref[...] = jnp.minimum(acc, E - 1)


def _route(x, wg_p, bg_p):
    return pl.pallas_call(
        _route_body,
        grid=(2, N // BN),
        in_specs=[
            pl.BlockSpec((BN, D), lambda p, i: ((1 - p) * i, 0)),
            pl.BlockSpec((D, LANES), lambda p, i: (0, 0)),
            pl.BlockSpec((1, LANES), lambda p, i: (0, 0)),
        ],
        out_specs=[
            pl.BlockSpec((BN, 1), lambda p, i: (i, 0)),
            pl.BlockSpec((1, LANES), lambda p, i: (0, 0)),
        ],
        out_shape=[
            jax.ShapeDtypeStruct((N, 1), i32),
            jax.ShapeDtypeStruct((1, LANES), i32),
        ],
        scratch_shapes=[
            pltpu.VMEM((N, 1), i32),
            pltpu.VMEM((N, 1), i32),
            pltpu.VMEM((1, LANES), f32),
        ],
    )(x, wg_p, bg_p)


def _sc_mesh():
    return plsc.VectorSubcoreMesh(core_axis_name="c", subcore_axis_name="s",
                                  num_cores=2, num_subcores=16)


NBUF = 3
NCHUNK = RPW // C


def _sc_scratch():
    st = []
    for _ in range(NBUF):
        st += [pltpu.VMEM((C,), i32), pltpu.VMEM((C, D), f32),
               pltpu.SemaphoreType.DMA, pltpu.SemaphoreType.DMA]
    return st


def _permute(x, slot):
    @functools.partial(
        pl.kernel,
        out_type=jax.ShapeDtypeStruct((PADDED_N, D), f32),
        mesh=_sc_mesh(),
        scratch_types=_sc_scratch(),
    )
    def body(x_hbm, slot_hbm, xs_hbm, *bufs):
        wid = lax.axis_index("s") * 2 + lax.axis_index("c")
        base = wid * RPW
        grp = [bufs[4 * b:4 * b + 4] for b in range(NBUF)]
        loads = [None] * NBUF
        scats = [None] * NBUF

        def load(j):
            b = j % NBUF
            idx_v, rows_v, sem_in, _ = grp[b]
            off = base + j * C
            pltpu.sync_copy(slot_hbm.at[pl.ds(off, C)], idx_v)
            loads[b] = pltpu.async_copy(x_hbm.at[pl.ds(off, C)], rows_v, sem_in)

        load(0)
        if NCHUNK > 1:
            load(1)
        for j in range(NCHUNK):
            b = j % NBUF
            idx_v, rows_v, _, sem_out = grp[b]
            loads[b].wait()
            scats[b] = pltpu.async_copy(rows_v, xs_hbm.at[idx_v], sem_out)
            nj = j + 2
            if nj < NCHUNK:
                nb = nj % NBUF
                if scats[nb] is not None:
                    scats[nb].wait()
                    scats[nb] = None
                load(nj)
        for b in range(NBUF):
            if scats[b] is not None:
                scats[b].wait()

    return body(x, slot)


def _unpermute(ys, slot):
    @functools.partial(
        pl.kernel,
        out_type=jax.ShapeDtypeStruct((N, D), f32),
        mesh=_sc_mesh(),
        scratch_types=_sc_scratch(),
    )
    def body(ys_hbm, slot_hbm, out_hbm, *bufs):
        wid = lax.axis_index("s") * 2 + lax.axis_index("c")
        base = wid * RPW
        grp = [bufs[4 * b:4 * b + 4] for b in range(NBUF)]
        gaths = [None] * NBUF
        stores = [None] * NBUF

        def gather(j):
            b = j % NBUF
            idx_v, rows_v, sem_in, _ = grp[b]
            off = base + j * C
            pltpu.sync_copy(slot_hbm.at[pl.ds(off, C)], idx_v)
            gaths[b] = pltpu.async_copy(ys_hbm.at[idx_v], rows_v, sem_in)

        gather(0)
        if NCHUNK > 1:
            gather(1)
        for j in range(NCHUNK):
            b = j % NBUF
            _, rows_v, _, sem_out = grp[b]
            gaths[b].wait()
            stores[b] = pltpu.async_copy(rows_v, out_hbm.at[pl.ds(base + j * C, C)],
                                         sem_out)
            nj = j + 2
            if nj < NCHUNK:
                nb = nj % NBUF
                if stores[nb] is not None:
                    stores[nb].wait()
                    stores[nb] = None
                gather(nj)
        for b in range(NBUF):
            if stores[b] is not None:
                stores[b].wait()

    return body(ys, slot)


def _moe_body(te_ref, x_ref, w0_ref, b0_ref, w1_ref, b1_ref, w2_ref, b2_ref,
              y_ref, w0b, w1b, w2b):
    i = pl.program_id(0)
    bf16 = jnp.bfloat16
    changed = jnp.logical_or(i == 0, te_ref[i] != te_ref[jnp.maximum(i - 1, 0)])

    @pl.when(changed)
    def _():
        w0b[...] = w0_ref[0].astype(bf16)
        w1b[...] = w1_ref[0].astype(bf16)
        w2b[...] = w2_ref[0].astype(bf16)

    h = jnp.dot(x_ref[...].astype(bf16), w0b[...], preferred_element_type=f32)
    h = jnp.maximum(h + b0_ref[0], 0.0).astype(bf16)
    h = jnp.dot(h, w1b[...], preferred_element_type=f32)
    h = jnp.maximum(h + b1_ref[0], 0.0).astype(bf16)
    y_ref[...] = jnp.dot(h, w2b[...], preferred_element_type=f32) + b2_ref[0]


def _moe(te, xs, W0, b0, W1, b1, W2, b2):
    grid_spec = pltpu.PrefetchScalarGridSpec(
        num_scalar_prefetch=1,
        grid=(NT,),
        in_specs=[
            pl.BlockSpec((T, D), lambda i, te: (i, 0)),
            pl.BlockSpec((1, D, D), lambda i, te: (te[i], 0, 0)),
            pl.BlockSpec((1, 1, D), lambda i, te: (te[i], 0, 0)),
            pl.BlockSpec((1, D, D), lambda i, te: (te[i], 0, 0)),
            pl.BlockSpec((1, 1, D), lambda i, te: (te[i], 0, 0)),
            pl.BlockSpec((1, D, D), lambda i, te: (te[i], 0, 0)),
            pl.BlockSpec((1, 1, D), lambda i, te: (te[i], 0, 0)),
        ],
        out_specs=pl.BlockSpec((T, D), lambda i, te: (i, 0)),
        scratch_shapes=[
            pltpu.VMEM((D, D), jnp.bfloat16),
            pltpu.VMEM((D, D), jnp.bfloat16),
            pltpu.VMEM((D, D), jnp.bfloat16),
        ],
    )
    return pl.pallas_call(
        _moe_body,
        grid_spec=grid_spec,
        out_shape=jax.ShapeDtypeStruct((PADDED_N, D), f32),
    )(te, xs, W0, b0, W1, b1, W2, b2)


def kernel(x, Wg, bg, W0, b0, W1, b1, W2, b2):
    wg_p = jnp.zeros((D, LANES), f32).at[:, :E].set(Wg)
    bg_p = jnp.full((1, LANES), -1e30, f32).at[0, :E].set(bg)
    slot2, te2 = _route(x, wg_p, bg_p)
    slot = slot2.reshape(N)
    te = te2.reshape(LANES)
    xs = _permute(x, slot)
    ys = _moe(te, xs, W0, b0.reshape(E, 1, D), W1, b1.reshape(E, 1, D),
              W2, b2.reshape(E, 1, D))
    return _unpermute(ys, slot)


# speedup vs baseline: 1.0368x; 1.0368x over previous
import functools

import jax
import jax.numpy as jnp
from jax import lax
from jax.experimental import pallas as pl
from jax.experimental.pallas import tpu as pltpu
from jax.experimental.pallas import tpu_sc as plsc

N = 8192
D = 1024
E = 8
LANES = 128
BN = 512
T = 256
NT = N // T + E
PADDED_N = NT * T

NW = 32
RPW = N // NW
C = 32

f32 = jnp.float32
i32 = jnp.int32


def _route_body(x_ref, wg_ref, bg_ref, slot_ref, te_ref, idx_s, rank_s, cnt_s):
    p = pl.program_id(0)
    i = pl.program_id(1)

    @pl.when(p == 0)
    def _phase0():
        logits = jnp.dot(x_ref[...], wg_ref[...],
                         preferred_element_type=f32) + bg_ref[...]
        m = jnp.max(logits, axis=1, keepdims=True)
        lane = lax.broadcasted_iota(i32, logits.shape, 1)
        idxv = jnp.min(jnp.where(logits == m, lane, E), axis=1, keepdims=True)
        oh = (lane == idxv).astype(f32)
        r = lax.broadcasted_iota(i32, (BN, BN), 0)
        c = lax.broadcasted_iota(i32, (BN, BN), 1)
        ltri = (r > c).astype(f32)
        csum = jnp.dot(ltri, oh, preferred_element_type=f32)
        carry = jnp.where(i == 0, jnp.zeros((1, E), f32), cnt_s[...])
        rank = jnp.sum(oh * (csum + carry), axis=1, keepdims=True)
        idx_s[pl.ds(i * BN, BN), :] = idxv
        rank_s[pl.ds(i * BN, BN), :] = rank.astype(i32)
        cnt_s[...] = carry + jnp.sum(oh, axis=0, keepdims=True)

    @pl.when(p == 1)
    def _phase1():
        starts, ends = [], []
        cum = i32(0)
        for e in range(E):
            cnt = cnt_s[0, e].astype(i32)
            padded = ((cnt + (T - 1)) // T) * T
            starts.append(cum)
            cum = cum + padded
            ends.append(cum)
        idxb = idx_s[pl.ds(i * BN, BN), :]
        slot = rank_s[pl.ds(i * BN, BN), :]
        for e in range(E):
            slot = slot + jnp.where(idxb == e, starts[e], 0)
        slot_ref[...] = slot

        @pl.when(i == 0)
        def _():
            t = lax.broadcasted_iota(i32, (1, LANES), 1) * T
            acc = jnp.zeros((1, LANES), i32)
            for e in range(E):
                acc = acc + (t >= ends[e]).astype(i32)
            te_ref[...] = jnp.minimum(acc, E - 1)


def _route(x, wg_p, bg_p):
    return pl.pallas_call(
        _route_body,
        grid=(2, N // BN),
        in_specs=[
            pl.BlockSpec((BN, D), lambda p, i: ((1 - p) * i, 0)),
            pl.BlockSpec((D, E), lambda p, i: (0, 0)),
            pl.BlockSpec((1, E), lambda p, i: (0, 0)),
        ],
        out_specs=[
            pl.BlockSpec((BN, 1), lambda p, i: (i, 0)),
            pl.BlockSpec((1, LANES), lambda p, i: (0, 0)),
        ],
        out_shape=[
            jax.ShapeDtypeStruct((N, 1), i32),
            jax.ShapeDtypeStruct((1, LANES), i32),
        ],
        scratch_shapes=[
            pltpu.VMEM((N, 1), i32),
            pltpu.VMEM((N, 1), i32),
            pltpu.VMEM((1, E), f32),
        ],
    )(x, wg_p, bg_p)


def _sc_mesh():
    return plsc.VectorSubcoreMesh(core_axis_name="c", subcore_axis_name="s",
                                  num_cores=2, num_subcores=16)


NBUF = 3
NCHUNK = RPW // C


def _sc_scratch():
    st = []
    for _ in range(NBUF):
        st += [pltpu.VMEM((C,), i32), pltpu.VMEM((C, D), f32),
               pltpu.SemaphoreType.DMA, pltpu.SemaphoreType.DMA]
    return st


def _permute(x, slot):
    @functools.partial(
        pl.kernel,
        out_type=jax.ShapeDtypeStruct((PADDED_N, D), f32),
        mesh=_sc_mesh(),
        scratch_types=_sc_scratch(),
    )
    def body(x_hbm, slot_hbm, xs_hbm, *bufs):
        wid = lax.axis_index("s") * 2 + lax.axis_index("c")
        base = wid * RPW
        grp = [bufs[4 * b:4 * b + 4] for b in range(NBUF)]
        loads = [None] * NBUF
        scats = [None] * NBUF

        def load(j):
            b = j % NBUF
            idx_v, rows_v, sem_in, _ = grp[b]
            off = base + j * C
            pltpu.sync_copy(slot_hbm.at[pl.ds(off, C)], idx_v)
            loads[b] = pltpu.async_copy(x_hbm.at[pl.ds(off, C)], rows_v, sem_in)

        load(0)
        if NCHUNK > 1:
            load(1)
        for j in range(NCHUNK):
            b = j % NBUF
            idx_v, rows_v, _, sem_out = grp[b]
            loads[b].wait()
            scats[b] = pltpu.async_copy(rows_v, xs_hbm.at[idx_v], sem_out)
            nj = j + 2
            if nj < NCHUNK:
                nb = nj % NBUF
                if scats[nb] is not None:
                    scats[nb].wait()
                    scats[nb] = None
                load(nj)
        for b in range(NBUF):
            if scats[b] is not None:
                scats[b].wait()

    return body(x, slot)


def _unpermute(ys, slot):
    @functools.partial(
        pl.kernel,
        out_type=jax.ShapeDtypeStruct((N, D), f32),
        mesh=_sc_mesh(),
        scratch_types=_sc_scratch(),
    )
    def body(ys_hbm, slot_hbm, out_hbm, *bufs):
        wid = lax.axis_index("s") * 2 + lax.axis_index("c")
        base = wid * RPW
        grp = [bufs[4 * b:4 * b + 4] for b in range(NBUF)]
        gaths = [None] * NBUF
        stores = [None] * NBUF

        def gather(j):
            b = j % NBUF
            idx_v, rows_v, sem_in, _ = grp[b]
            off = base + j * C
            pltpu.sync_copy(slot_hbm.at[pl.ds(off, C)], idx_v)
            gaths[b] = pltpu.async_copy(ys_hbm.at[idx_v], rows_v, sem_in)

        gather(0)
        if NCHUNK > 1:
            gather(1)
        for j in range(NCHUNK):
            b = j % NBUF
            _, rows_v, _, sem_out = grp[b]
            gaths[b].wait()
            stores[b] = pltpu.async_copy(rows_v, out_hbm.at[pl.ds(base + j * C, C)],
                                         sem_out)
            nj = j + 2
            if nj < NCHUNK:
                nb = nj % NBUF
                if stores[nb] is not None:
                    stores[nb].wait()
                    stores[nb] = None
                gather(nj)
        for b in range(NBUF):
            if stores[b] is not None:
                stores[b].wait()

    return body(ys, slot)


def _moe_body(te_ref, x_ref, w0_ref, b0_ref, w1_ref, b1_ref, w2_ref, b2_ref,
              y_ref):
    h = jnp.dot(x_ref[...], w0_ref[0], preferred_element_type=f32)
    h = jnp.maximum(h + b0_ref[0], 0.0)
    h = jnp.dot(h, w1_ref[0], preferred_element_type=f32)
    h = jnp.maximum(h + b1_ref[0], 0.0)
    y_ref[...] = jnp.dot(h, w2_ref[0], preferred_element_type=f32) + b2_ref[0]


def _moe(te, xs, W0, b0, W1, b1, W2, b2):
    grid_spec = pltpu.PrefetchScalarGridSpec(
        num_scalar_prefetch=1,
        grid=(NT,),
        in_specs=[
            pl.BlockSpec((T, D), lambda i, te: (i, 0)),
            pl.BlockSpec((1, D, D), lambda i, te: (te[i], 0, 0)),
            pl.BlockSpec((1, 1, D), lambda i, te: (te[i], 0, 0)),
            pl.BlockSpec((1, D, D), lambda i, te: (te[i], 0, 0)),
            pl.BlockSpec((1, 1, D), lambda i, te: (te[i], 0, 0)),
            pl.BlockSpec((1, D, D), lambda i, te: (te[i], 0, 0)),
            pl.BlockSpec((1, 1, D), lambda i, te: (te[i], 0, 0)),
        ],
        out_specs=pl.BlockSpec((T, D), lambda i, te: (i, 0)),
    )
    return pl.pallas_call(
        _moe_body,
        grid_spec=grid_spec,
        out_shape=jax.ShapeDtypeStruct((PADDED_N, D), f32),
    )(te, xs, W0, b0, W1, b1, W2, b2)


def kernel(x, Wg, bg, W0, b0, W1, b1, W2, b2):
    slot2, te2 = _route(x, Wg, bg.reshape(1, E))
    slot = slot2.reshape(N)
    te = te2.reshape(LANES)
    xs = _permute(x, slot)
    ys = _moe(te, xs, W0, b0.reshape(E, 1, D), W1, b1.reshape(E, 1, D),
              W2, b2.reshape(E, 1, D))
    return _unpermute(ys, slot)


# speedup vs baseline: 1.0928x; 1.0539x over previous
import functools

import jax
import jax.numpy as jnp
from jax import lax
from jax.experimental import pallas as pl
from jax.experimental.pallas import tpu as pltpu
from jax.experimental.pallas import tpu_sc as plsc

N = 8192
D = 1024
E = 8
LANES = 128
BN = 512
T = 256
NT = N // T + E
PADDED_N = NT * T

NW = 32
RPW = N // NW
C = 32

f32 = jnp.float32
i32 = jnp.int32


def _route_body(x_ref, wg_ref, bg_ref, slot_ref, te_ref, nxe_ref, par_ref,
                idx_s, rank_s, cnt_s):
    p = pl.program_id(0)
    i = pl.program_id(1)

    @pl.when(p == 0)
    def _phase0():
        logits = jnp.dot(x_ref[...], wg_ref[...],
                         preferred_element_type=f32) + bg_ref[...]
        m = jnp.max(logits, axis=1, keepdims=True)
        lane = lax.broadcasted_iota(i32, logits.shape, 1)
        idxv = jnp.min(jnp.where(logits == m, lane, E), axis=1, keepdims=True)
        oh = (lane == idxv).astype(f32)
        r = lax.broadcasted_iota(i32, (BN, BN), 0)
        c = lax.broadcasted_iota(i32, (BN, BN), 1)
        ltri = (r > c).astype(f32)
        csum = jnp.dot(ltri, oh, preferred_element_type=f32)
        carry = jnp.where(i == 0, jnp.zeros((1, E), f32), cnt_s[...])
        rank = jnp.sum(oh * (csum + carry), axis=1, keepdims=True)
        idx_s[pl.ds(i * BN, BN), :] = idxv
        rank_s[pl.ds(i * BN, BN), :] = rank.astype(i32)
        cnt_s[...] = carry + jnp.sum(oh, axis=0, keepdims=True)

    @pl.when(p == 1)
    def _phase1():
        starts, ends, pcs = [], [], []
        cum = i32(0)
        for e in range(E):
            cnt = cnt_s[0, e].astype(i32)
            padded = ((cnt + (T - 1)) // T) * T
            starts.append(cum)
            pcs.append(padded)
            cum = cum + padded
            ends.append(cum)
        idxb = idx_s[pl.ds(i * BN, BN), :]
        slot = rank_s[pl.ds(i * BN, BN), :]
        for e in range(E):
            slot = slot + jnp.where(idxb == e, starts[e], 0)
        slot_ref[...] = slot

        @pl.when(i == 0)
        def _():
            t = lax.broadcasted_iota(i32, (1, LANES), 1) * T
            acc = jnp.zeros((1, LANES), i32)
            lp = i32(0)
            for e in range(E):
                acc = acc + (t >= ends[e]).astype(i32)
                lp = jnp.where(pcs[e] > 0, e, lp)
            te = jnp.minimum(acc, lp)
            te_ref[...] = te
            nxe = te
            for e in reversed(range(E)):
                nxe = jnp.where((pcs[e] > 0) & (e > te), e, nxe)
            nxe_ref[...] = nxe
            gidx = jnp.zeros((1, LANES), i32)
            for e in range(E):
                gidx = gidx + jnp.where((pcs[e] > 0) & (e <= te), 1, 0)
            par_ref[...] = (gidx - 1) % 2


def _route(x, wg_p, bg_p):
    return pl.pallas_call(
        _route_body,
        grid=(2, N // BN),
        in_specs=[
            pl.BlockSpec((BN, D), lambda p, i: ((1 - p) * i, 0)),
            pl.BlockSpec((D, E), lambda p, i: (0, 0)),
            pl.BlockSpec((1, E), lambda p, i: (0, 0)),
        ],
        out_specs=[
            pl.BlockSpec((BN, 1), lambda p, i: (i, 0)),
            pl.BlockSpec((1, LANES), lambda p, i: (0, 0)),
            pl.BlockSpec((1, LANES), lambda p, i: (0, 0)),
            pl.BlockSpec((1, LANES), lambda p, i: (0, 0)),
        ],
        out_shape=[
            jax.ShapeDtypeStruct((N, 1), i32),
            jax.ShapeDtypeStruct((1, LANES), i32),
            jax.ShapeDtypeStruct((1, LANES), i32),
            jax.ShapeDtypeStruct((1, LANES), i32),
        ],
        scratch_shapes=[
            pltpu.VMEM((N, 1), i32),
            pltpu.VMEM((N, 1), i32),
            pltpu.VMEM((1, E), f32),
        ],
    )(x, wg_p, bg_p)


def _sc_mesh():
    return plsc.VectorSubcoreMesh(core_axis_name="c", subcore_axis_name="s",
                                  num_cores=2, num_subcores=16)


NBUF = 3
NCHUNK = RPW // C


def _sc_scratch():
    st = []
    for _ in range(NBUF):
        st += [pltpu.VMEM((C,), i32), pltpu.VMEM((C, D), f32),
               pltpu.SemaphoreType.DMA, pltpu.SemaphoreType.DMA]
    return st


def _permute(x, slot):
    @functools.partial(
        pl.kernel,
        out_type=jax.ShapeDtypeStruct((PADDED_N, D), f32),
        mesh=_sc_mesh(),
        scratch_types=_sc_scratch(),
    )
    def body(x_hbm, slot_hbm, xs_hbm, *bufs):
        wid = lax.axis_index("s") * 2 + lax.axis_index("c")
        base = wid * RPW
        grp = [bufs[4 * b:4 * b + 4] for b in range(NBUF)]
        loads = [None] * NBUF
        scats = [None] * NBUF

        def load(j):
            b = j % NBUF
            idx_v, rows_v, sem_in, _ = grp[b]
            off = base + j * C
            pltpu.sync_copy(slot_hbm.at[pl.ds(off, C)], idx_v)
            loads[b] = pltpu.async_copy(x_hbm.at[pl.ds(off, C)], rows_v, sem_in)

        load(0)
        if NCHUNK > 1:
            load(1)
        for j in range(NCHUNK):
            b = j % NBUF
            idx_v, rows_v, _, sem_out = grp[b]
            loads[b].wait()
            scats[b] = pltpu.async_copy(rows_v, xs_hbm.at[idx_v], sem_out)
            nj = j + 2
            if nj < NCHUNK:
                nb = nj % NBUF
                if scats[nb] is not None:
                    scats[nb].wait()
                    scats[nb] = None
                load(nj)
        for b in range(NBUF):
            if scats[b] is not None:
                scats[b].wait()

    return body(x, slot)


def _unpermute(ys, slot):
    @functools.partial(
        pl.kernel,
        out_type=jax.ShapeDtypeStruct((N, D), f32),
        mesh=_sc_mesh(),
        scratch_types=_sc_scratch(),
    )
    def body(ys_hbm, slot_hbm, out_hbm, *bufs):
        wid = lax.axis_index("s") * 2 + lax.axis_index("c")
        base = wid * RPW
        grp = [bufs[4 * b:4 * b + 4] for b in range(NBUF)]
        gaths = [None] * NBUF
        stores = [None] * NBUF

        def gather(j):
            b = j % NBUF
            idx_v, rows_v, sem_in, _ = grp[b]
            off = base + j * C
            pltpu.sync_copy(slot_hbm.at[pl.ds(off, C)], idx_v)
            gaths[b] = pltpu.async_copy(ys_hbm.at[idx_v], rows_v, sem_in)

        gather(0)
        if NCHUNK > 1:
            gather(1)
        for j in range(NCHUNK):
            b = j % NBUF
            _, rows_v, _, sem_out = grp[b]
            gaths[b].wait()
            stores[b] = pltpu.async_copy(rows_v, out_hbm.at[pl.ds(base + j * C, C)],
                                         sem_out)
            nj = j + 2
            if nj < NCHUNK:
                nb = nj % NBUF
                if stores[nb] is not None:
                    stores[nb].wait()
                    stores[nb] = None
                gather(nj)
        for b in range(NBUF):
            if stores[b] is not None:
                stores[b].wait()

    return body(ys, slot)


def _moe_body(te_ref, nxe_ref, par_ref, x_ref, b0_ref, b1_ref, b2_ref,
              w0_hbm, w1_hbm, w2_hbm, y_ref, w0b, w1b, w2b, sems):
    i = pl.program_id(0)
    cur = te_ref[i]
    par = par_ref[i]
    nxe = nxe_ref[i]
    changed = jnp.logical_or(i == 0, te_ref[i] != te_ref[jnp.maximum(i - 1, 0)])

    @pl.when(i == 0)
    def _():
        pltpu.make_async_copy(w0_hbm.at[cur], w0b.at[0], sems.at[0]).start()
        pltpu.make_async_copy(w1_hbm.at[cur], w1b.at[0], sems.at[0]).start()
        pltpu.make_async_copy(w2_hbm.at[cur], w2b.at[0], sems.at[0]).start()

    @pl.when(changed)
    def _():
        pltpu.make_async_copy(w0_hbm.at[cur], w0b.at[par], sems.at[par]).wait()
        pltpu.make_async_copy(w1_hbm.at[cur], w1b.at[par], sems.at[par]).wait()
        pltpu.make_async_copy(w2_hbm.at[cur], w2b.at[par], sems.at[par]).wait()

    @pl.when(jnp.logical_and(changed, nxe != cur))
    def _():
        nslot = 1 - par
        pltpu.make_async_copy(w0_hbm.at[nxe], w0b.at[nslot], sems.at[nslot]).start()
        pltpu.make_async_copy(w1_hbm.at[nxe], w1b.at[nslot], sems.at[nslot]).start()
        pltpu.make_async_copy(w2_hbm.at[nxe], w2b.at[nslot], sems.at[nslot]).start()

    h = jnp.dot(x_ref[...], w0b[par], preferred_element_type=f32)
    h = jnp.maximum(h + b0_ref[0], 0.0)
    h = jnp.dot(h, w1b[par], preferred_element_type=f32)
    h = jnp.maximum(h + b1_ref[0], 0.0)
    y_ref[...] = jnp.dot(h, w2b[par], preferred_element_type=f32) + b2_ref[0]


def _moe(te, nxe, par, xs, W0, b0, W1, b1, W2, b2):
    grid_spec = pltpu.PrefetchScalarGridSpec(
        num_scalar_prefetch=3,
        grid=(NT,),
        in_specs=[
            pl.BlockSpec((T, D), lambda i, te, nxe, par: (i, 0)),
            pl.BlockSpec((1, 1, D), lambda i, te, nxe, par: (te[i], 0, 0)),
            pl.BlockSpec((1, 1, D), lambda i, te, nxe, par: (te[i], 0, 0)),
            pl.BlockSpec((1, 1, D), lambda i, te, nxe, par: (te[i], 0, 0)),
            pl.BlockSpec(memory_space=pltpu.MemorySpace.HBM),
            pl.BlockSpec(memory_space=pltpu.MemorySpace.HBM),
            pl.BlockSpec(memory_space=pltpu.MemorySpace.HBM),
        ],
        out_specs=pl.BlockSpec((T, D), lambda i, te, nxe, par: (i, 0)),
        scratch_shapes=[
            pltpu.VMEM((2, D, D), f32),
            pltpu.VMEM((2, D, D), f32),
            pltpu.VMEM((2, D, D), f32),
            pltpu.SemaphoreType.DMA((2,)),
        ],
    )
    return pl.pallas_call(
        _moe_body,
        grid_spec=grid_spec,
        out_shape=jax.ShapeDtypeStruct((PADDED_N, D), f32),
    )(te, nxe, par, xs, b0, b1, b2, W0, W1, W2)


def kernel(x, Wg, bg, W0, b0, W1, b1, W2, b2):
    slot2, te2, nxe2, par2 = _route(x, Wg, bg.reshape(1, E))
    slot = slot2.reshape(N)
    xs = _permute(x, slot)
    ys = _moe(te2.reshape(LANES), nxe2.reshape(LANES), par2.reshape(LANES),
              xs, W0, b0.reshape(E, 1, D), W1, b1.reshape(E, 1, D),
              W2, b2.reshape(E, 1, D))
    return _unpermute(ys, slot)


# speedup vs baseline: 1.1283x; 1.0326x over previous
import functools

import jax
import jax.numpy as jnp
from jax import lax
from jax.experimental import pallas as pl
from jax.experimental.pallas import tpu as pltpu
from jax.experimental.pallas import tpu_sc as plsc

N = 8192
D = 1024
E = 8
LANES = 128
BN = 512
T = 256
NT = N // T + E
PADDED_N = NT * T

NW = 32
RPW = N // NW
C = 32

f32 = jnp.float32
i32 = jnp.int32


def _route_body(x_ref, wg_ref, bg_ref, slot_ref, te_ref, nxe_ref, par_ref,
                idx_s, rank_s, cnt_s):
    p = pl.program_id(0)
    i = pl.program_id(1)

    @pl.when(p == 0)
    def _phase0():
        logits = lax.dot_general(x_ref[...], wg_ref[...],
                                 (((1,), (1,)), ((), ())),
                                 preferred_element_type=f32) + bg_ref[...]
        m = jnp.max(logits, axis=1, keepdims=True)
        lane = lax.broadcasted_iota(i32, logits.shape, 1)
        idxv = jnp.min(jnp.where(logits == m, lane, E), axis=1, keepdims=True)
        oh = (lane == idxv).astype(f32)
        r = lax.broadcasted_iota(i32, (BN, BN), 0)
        c = lax.broadcasted_iota(i32, (BN, BN), 1)
        ltri = (r > c).astype(f32)
        csum = jnp.dot(ltri, oh, preferred_element_type=f32)
        carry = jnp.where(i == 0, jnp.zeros((1, E), f32), cnt_s[...])
        rank = jnp.sum(oh * (csum + carry), axis=1, keepdims=True)
        idx_s[pl.ds(i * BN, BN), :] = idxv
        rank_s[pl.ds(i * BN, BN), :] = rank.astype(i32)
        cnt_s[...] = carry + jnp.sum(oh, axis=0, keepdims=True)

    @pl.when(p == 1)
    def _phase1():
        starts, ends, pcs = [], [], []
        cum = i32(0)
        for e in range(E):
            cnt = cnt_s[0, e].astype(i32)
            padded = ((cnt + (T - 1)) // T) * T
            starts.append(cum)
            pcs.append(padded)
            cum = cum + padded
            ends.append(cum)
        idxb = idx_s[pl.ds(i * BN, BN), :]
        slot = rank_s[pl.ds(i * BN, BN), :]
        for e in range(E):
            slot = slot + jnp.where(idxb == e, starts[e], 0)
        slot_ref[...] = slot.reshape(BN)

        @pl.when(i == 0)
        def _():
            t = lax.broadcasted_iota(i32, (1, LANES), 1) * T
            acc = jnp.zeros((1, LANES), i32)
            lp = i32(0)
            for e in range(E):
                acc = acc + (t >= ends[e]).astype(i32)
                lp = jnp.where(pcs[e] > 0, e, lp)
            te = jnp.minimum(acc, lp)
            te_ref[...] = te
            nxe = te
            for e in reversed(range(E)):
                nxe = jnp.where((pcs[e] > 0) & (e > te), e, nxe)
            nxe_ref[...] = nxe
            gidx = jnp.zeros((1, LANES), i32)
            for e in range(E):
                gidx = gidx + jnp.where((pcs[e] > 0) & (e <= te), 1, 0)
            par_ref[...] = (gidx - 1) % 2


def _route(x, wg_p, bg_p):
    return pl.pallas_call(
        _route_body,
        grid=(2, N // BN),
        in_specs=[
            pl.BlockSpec((BN, D), lambda p, i: ((1 - p) * i, 0)),
            pl.BlockSpec((E, D), lambda p, i: (0, 0)),
            pl.BlockSpec((1, E), lambda p, i: (0, 0)),
        ],
        out_specs=[
            pl.BlockSpec((BN,), lambda p, i: (i,)),
            pl.BlockSpec((1, LANES), lambda p, i: (0, 0)),
            pl.BlockSpec((1, LANES), lambda p, i: (0, 0)),
            pl.BlockSpec((1, LANES), lambda p, i: (0, 0)),
        ],
        out_shape=[
            jax.ShapeDtypeStruct((N,), i32),
            jax.ShapeDtypeStruct((1, LANES), i32),
            jax.ShapeDtypeStruct((1, LANES), i32),
            jax.ShapeDtypeStruct((1, LANES), i32),
        ],
        scratch_shapes=[
            pltpu.VMEM((N, 1), i32),
            pltpu.VMEM((N, 1), i32),
            pltpu.VMEM((1, E), f32),
        ],
    )(x, wg_p, bg_p)


def _sc_mesh():
    return plsc.VectorSubcoreMesh(core_axis_name="c", subcore_axis_name="s",
                                  num_cores=2, num_subcores=16)


NBUF = 3
NCHUNK = RPW // C


def _sc_scratch():
    st = []
    for _ in range(NBUF):
        st += [pltpu.VMEM((C,), i32), pltpu.VMEM((C, D), f32),
               pltpu.SemaphoreType.DMA, pltpu.SemaphoreType.DMA]
    return st


def _permute(x, slot):
    @functools.partial(
        pl.kernel,
        out_type=jax.ShapeDtypeStruct((PADDED_N, D), f32),
        mesh=_sc_mesh(),
        scratch_types=_sc_scratch(),
    )
    def body(x_hbm, slot_hbm, xs_hbm, *bufs):
        wid = lax.axis_index("s") * 2 + lax.axis_index("c")
        base = wid * RPW
        grp = [bufs[4 * b:4 * b + 4] for b in range(NBUF)]
        loads = [None] * NBUF
        scats = [None] * NBUF

        def load(j):
            b = j % NBUF
            idx_v, rows_v, sem_in, _ = grp[b]
            off = base + j * C
            pltpu.sync_copy(slot_hbm.at[pl.ds(off, C)], idx_v)
            loads[b] = pltpu.async_copy(x_hbm.at[pl.ds(off, C)], rows_v, sem_in)

        load(0)
        if NCHUNK > 1:
            load(1)
        for j in range(NCHUNK):
            b = j % NBUF
            idx_v, rows_v, _, sem_out = grp[b]
            loads[b].wait()
            scats[b] = pltpu.async_copy(rows_v, xs_hbm.at[idx_v], sem_out)
            nj = j + 2
            if nj < NCHUNK:
                nb = nj % NBUF
                if scats[nb] is not None:
                    scats[nb].wait()
                    scats[nb] = None
                load(nj)
        for b in range(NBUF):
            if scats[b] is not None:
                scats[b].wait()

    return body(x, slot)


def _unpermute(ys, slot):
    @functools.partial(
        pl.kernel,
        out_type=jax.ShapeDtypeStruct((N, D), f32),
        mesh=_sc_mesh(),
        scratch_types=_sc_scratch(),
    )
    def body(ys_hbm, slot_hbm, out_hbm, *bufs):
        wid = lax.axis_index("s") * 2 + lax.axis_index("c")
        base = wid * RPW
        grp = [bufs[4 * b:4 * b + 4] for b in range(NBUF)]
        gaths = [None] * NBUF
        stores = [None] * NBUF

        def gather(j):
            b = j % NBUF
            idx_v, rows_v, sem_in, _ = grp[b]
            off = base + j * C
            pltpu.sync_copy(slot_hbm.at[pl.ds(off, C)], idx_v)
            gaths[b] = pltpu.async_copy(ys_hbm.at[idx_v], rows_v, sem_in)

        gather(0)
        if NCHUNK > 1:
            gather(1)
        for j in range(NCHUNK):
            b = j % NBUF
            _, rows_v, _, sem_out = grp[b]
            gaths[b].wait()
            stores[b] = pltpu.async_copy(rows_v, out_hbm.at[pl.ds(base + j * C, C)],
                                         sem_out)
            nj = j + 2
            if nj < NCHUNK:
                nb = nj % NBUF
                if stores[nb] is not None:
                    stores[nb].wait()
                    stores[nb] = None
                gather(nj)
        for b in range(NBUF):
            if stores[b] is not None:
                stores[b].wait()

    return body(ys, slot)


def _moe_body(te_ref, nxe_ref, par_ref, x_ref, b0_ref, b1_ref, b2_ref,
              w0_hbm, w1_hbm, w2_hbm, y_ref, w0b, w1b, w2b, sems):
    i = pl.program_id(0)
    cur = te_ref[i]
    par = par_ref[i]
    nxe = nxe_ref[i]
    changed = jnp.logical_or(i == 0, te_ref[i] != te_ref[jnp.maximum(i - 1, 0)])

    @pl.when(i == 0)
    def _():
        pltpu.make_async_copy(w0_hbm.at[cur], w0b.at[0], sems.at[0]).start()
        pltpu.make_async_copy(w1_hbm.at[cur], w1b.at[0], sems.at[0]).start()
        pltpu.make_async_copy(w2_hbm.at[cur], w2b.at[0], sems.at[0]).start()

    @pl.when(changed)
    def _():
        pltpu.make_async_copy(w0_hbm.at[cur], w0b.at[par], sems.at[par]).wait()
        pltpu.make_async_copy(w1_hbm.at[cur], w1b.at[par], sems.at[par]).wait()
        pltpu.make_async_copy(w2_hbm.at[cur], w2b.at[par], sems.at[par]).wait()

    @pl.when(jnp.logical_and(changed, nxe != cur))
    def _():
        nslot = 1 - par
        pltpu.make_async_copy(w0_hbm.at[nxe], w0b.at[nslot], sems.at[nslot]).start()
        pltpu.make_async_copy(w1_hbm.at[nxe], w1b.at[nslot], sems.at[nslot]).start()
        pltpu.make_async_copy(w2_hbm.at[nxe], w2b.at[nslot], sems.at[nslot]).start()

    h = jnp.dot(x_ref[...], w0b[par], preferred_element_type=f32)
    h = jnp.maximum(h + b0_ref[0], 0.0)
    h = jnp.dot(h, w1b[par], preferred_element_type=f32)
    h = jnp.maximum(h + b1_ref[0], 0.0)
    y_ref[...] = jnp.dot(h, w2b[par], preferred_element_type=f32) + b2_ref[0]


def _moe(te, nxe, par, xs, W0, b0, W1, b1, W2, b2):
    grid_spec = pltpu.PrefetchScalarGridSpec(
        num_scalar_prefetch=3,
        grid=(NT,),
        in_specs=[
            pl.BlockSpec((T, D), lambda i, te, nxe, par: (i, 0)),
            pl.BlockSpec((1, 1, D), lambda i, te, nxe, par: (te[i], 0, 0)),
            pl.BlockSpec((1, 1, D), lambda i, te, nxe, par: (te[i], 0, 0)),
            pl.BlockSpec((1, 1, D), lambda i, te, nxe, par: (te[i], 0, 0)),
            pl.BlockSpec(memory_space=pltpu.MemorySpace.HBM),
            pl.BlockSpec(memory_space=pltpu.MemorySpace.HBM),
            pl.BlockSpec(memory_space=pltpu.MemorySpace.HBM),
        ],
        out_specs=pl.BlockSpec((T, D), lambda i, te, nxe, par: (i, 0)),
        scratch_shapes=[
            pltpu.VMEM((2, D, D), f32),
            pltpu.VMEM((2, D, D), f32),
            pltpu.VMEM((2, D, D), f32),
            pltpu.SemaphoreType.DMA((2,)),
        ],
    )
    return pl.pallas_call(
        _moe_body,
        grid_spec=grid_spec,
        out_shape=jax.ShapeDtypeStruct((PADDED_N, D), f32),
    )(te, nxe, par, xs, b0, b1, b2, W0, W1, W2)


def kernel(x, Wg, bg, W0, b0, W1, b1, W2, b2):
    slot, te2, nxe2, par2 = _route(x, jnp.swapaxes(Wg, 0, 1), bg.reshape(1, E))
    xs = _permute(x, slot)
    ys = _moe(te2.reshape(LANES), nxe2.reshape(LANES), par2.reshape(LANES),
              xs, W0, b0.reshape(E, 1, D), W1, b1.reshape(E, 1, D),
              W2, b2.reshape(E, 1, D))
    return _unpermute(ys, slot)


# speedup vs baseline: 1.1631x; 1.0308x over previous
import functools

import jax
import jax.numpy as jnp
from jax import lax
from jax.experimental import pallas as pl
from jax.experimental.pallas import tpu as pltpu
from jax.experimental.pallas import tpu_sc as plsc

N = 8192
D = 1024
E = 8
LANES = 128
BN = 512
T = 256
NT = N // T + E
PADDED_N = NT * T

NW = 32
RPW = N // NW
C = 32
DP = D // 2

f32 = jnp.float32
i32 = jnp.int32


def _route_body(x_ref, wg_ref, bg_ref, slot_ref, te_ref, nxe_ref, par_ref,
                xp_ref, idx_s, rank_s, cnt_s):
    p = pl.program_id(0)
    i = pl.program_id(1)

    @pl.when(p == 0)
    def _phase0():
        logits = lax.dot_general(x_ref[...], wg_ref[...],
                                 (((1,), (1,)), ((), ())),
                                 preferred_element_type=f32) + bg_ref[...]
        m = jnp.max(logits, axis=1, keepdims=True)
        lane = lax.broadcasted_iota(i32, logits.shape, 1)
        idxv = jnp.min(jnp.where(logits == m, lane, E), axis=1, keepdims=True)
        oh = (lane == idxv).astype(f32)
        r = lax.broadcasted_iota(i32, (BN, BN), 0)
        c = lax.broadcasted_iota(i32, (BN, BN), 1)
        ltri = (r > c).astype(f32)
        csum = jnp.dot(ltri, oh, preferred_element_type=f32)
        carry = jnp.where(i == 0, jnp.zeros((1, E), f32), cnt_s[...])
        rank = jnp.sum(oh * (csum + carry), axis=1, keepdims=True)
        idx_s[pl.ds(i * BN, BN), :] = idxv
        rank_s[pl.ds(i * BN, BN), :] = rank.astype(i32)
        cnt_s[...] = carry + jnp.sum(oh, axis=0, keepdims=True)
        xb = x_ref[...]
        bl = lax.bitcast_convert_type(
            xb[:, :DP].astype(jnp.bfloat16), jnp.uint16).astype(i32)
        bh = lax.bitcast_convert_type(
            xb[:, DP:].astype(jnp.bfloat16), jnp.uint16).astype(i32)
        xp_ref[...] = bl | (bh << 16)

    @pl.when(p == 1)
    def _phase1():
        starts, ends, pcs = [], [], []
        cum = i32(0)
        for e in range(E):
            cnt = cnt_s[0, e].astype(i32)
            padded = ((cnt + (T - 1)) // T) * T
            starts.append(cum)
            pcs.append(padded)
            cum = cum + padded
            ends.append(cum)
        idxb = idx_s[pl.ds(i * BN, BN), :]
        slot = rank_s[pl.ds(i * BN, BN), :]
        for e in range(E):
            slot = slot + jnp.where(idxb == e, starts[e], 0)
        slot_ref[...] = slot.reshape(BN)

        @pl.when(i == 0)
        def _():
            t = lax.broadcasted_iota(i32, (1, LANES), 1) * T
            acc = jnp.zeros((1, LANES), i32)
            lp = i32(0)
            for e in range(E):
                acc = acc + (t >= ends[e]).astype(i32)
                lp = jnp.where(pcs[e] > 0, e, lp)
            te = jnp.minimum(acc, lp)
            te_ref[...] = te
            nxe = te
            for e in reversed(range(E)):
                nxe = jnp.where((pcs[e] > 0) & (e > te), e, nxe)
            nxe_ref[...] = nxe
            gidx = jnp.zeros((1, LANES), i32)
            for e in range(E):
                gidx = gidx + jnp.where((pcs[e] > 0) & (e <= te), 1, 0)
            par_ref[...] = (gidx - 1) % 2


def _route(x, wg_p, bg_p):
    return pl.pallas_call(
        _route_body,
        grid=(2, N // BN),
        in_specs=[
            pl.BlockSpec((BN, D), lambda p, i: ((1 - p) * i, 0)),
            pl.BlockSpec((E, D), lambda p, i: (0, 0)),
            pl.BlockSpec((1, E), lambda p, i: (0, 0)),
        ],
        out_specs=[
            pl.BlockSpec((BN,), lambda p, i: (i,)),
            pl.BlockSpec((1, LANES), lambda p, i: (0, 0)),
            pl.BlockSpec((1, LANES), lambda p, i: (0, 0)),
            pl.BlockSpec((1, LANES), lambda p, i: (0, 0)),
            pl.BlockSpec((BN, DP),
                         lambda p, i: ((1 - p) * i + p * (N // BN), 0)),
        ],
        out_shape=[
            jax.ShapeDtypeStruct((N,), i32),
            jax.ShapeDtypeStruct((1, LANES), i32),
            jax.ShapeDtypeStruct((1, LANES), i32),
            jax.ShapeDtypeStruct((1, LANES), i32),
            jax.ShapeDtypeStruct((N + BN, DP), i32),
        ],
        scratch_shapes=[
            pltpu.VMEM((N, 1), i32),
            pltpu.VMEM((N, 1), i32),
            pltpu.VMEM((1, E), f32),
        ],
    )(x, wg_p, bg_p)


def _sc_mesh():
    return plsc.VectorSubcoreMesh(core_axis_name="c", subcore_axis_name="s",
                                  num_cores=2, num_subcores=16)


NBUF = 3
NCHUNK = RPW // C


def _sc_scratch(width, dtype):
    st = []
    for _ in range(NBUF):
        st += [pltpu.VMEM((C,), i32), pltpu.VMEM((C, width), dtype),
               pltpu.SemaphoreType.DMA, pltpu.SemaphoreType.DMA]
    return st


def _permute(x, slot):
    @functools.partial(
        pl.kernel,
        out_type=jax.ShapeDtypeStruct((PADDED_N, DP), i32),
        mesh=_sc_mesh(),
        scratch_types=_sc_scratch(DP, i32),
    )
    def body(x_hbm, slot_hbm, xs_hbm, *bufs):
        wid = lax.axis_index("s") * 2 + lax.axis_index("c")
        base = wid * RPW
        grp = [bufs[4 * b:4 * b + 4] for b in range(NBUF)]
        loads = [None] * NBUF
        scats = [None] * NBUF

        def load(j):
            b = j % NBUF
            idx_v, rows_v, sem_in, _ = grp[b]
            off = base + j * C
            pltpu.sync_copy(slot_hbm.at[pl.ds(off, C)], idx_v)
            loads[b] = pltpu.async_copy(x_hbm.at[pl.ds(off, C)], rows_v, sem_in)

        load(0)
        if NCHUNK > 1:
            load(1)
        for j in range(NCHUNK):
            b = j % NBUF
            idx_v, rows_v, _, sem_out = grp[b]
            loads[b].wait()
            scats[b] = pltpu.async_copy(rows_v, xs_hbm.at[idx_v], sem_out)
            nj = j + 2
            if nj < NCHUNK:
                nb = nj % NBUF
                if scats[nb] is not None:
                    scats[nb].wait()
                    scats[nb] = None
                load(nj)
        for b in range(NBUF):
            if scats[b] is not None:
                scats[b].wait()

    return body(x, slot)


def _unpermute(ys, slot):
    @functools.partial(
        pl.kernel,
        out_type=jax.ShapeDtypeStruct((N, D), f32),
        mesh=_sc_mesh(),
        scratch_types=_sc_scratch(D, f32),
    )
    def body(ys_hbm, slot_hbm, out_hbm, *bufs):
        wid = lax.axis_index("s") * 2 + lax.axis_index("c")
        base = wid * RPW
        grp = [bufs[4 * b:4 * b + 4] for b in range(NBUF)]
        gaths = [None] * NBUF
        stores = [None] * NBUF

        def gather(j):
            b = j % NBUF
            idx_v, rows_v, sem_in, _ = grp[b]
            off = base + j * C
            pltpu.sync_copy(slot_hbm.at[pl.ds(off, C)], idx_v)
            gaths[b] = pltpu.async_copy(ys_hbm.at[idx_v], rows_v, sem_in)

        gather(0)
        if NCHUNK > 1:
            gather(1)
        for j in range(NCHUNK):
            b = j % NBUF
            _, rows_v, _, sem_out = grp[b]
            gaths[b].wait()
            stores[b] = pltpu.async_copy(rows_v, out_hbm.at[pl.ds(base + j * C, C)],
                                         sem_out)
            nj = j + 2
            if nj < NCHUNK:
                nb = nj % NBUF
                if stores[nb] is not None:
                    stores[nb].wait()
                    stores[nb] = None
                gather(nj)
        for b in range(NBUF):
            if stores[b] is not None:
                stores[b].wait()

    return body(ys, slot)


def _moe_body(te_ref, nxe_ref, par_ref, x_ref, b0_ref, b1_ref, b2_ref,
              w0_hbm, w1_hbm, w2_hbm, y_ref, w0b, w1b, w2b, sems):
    i = pl.program_id(0)
    cur = te_ref[i]
    par = par_ref[i]
    nxe = nxe_ref[i]
    changed = jnp.logical_or(i == 0, te_ref[i] != te_ref[jnp.maximum(i - 1, 0)])

    @pl.when(i == 0)
    def _():
        pltpu.make_async_copy(w0_hbm.at[cur], w0b.at[0], sems.at[0]).start()
        pltpu.make_async_copy(w1_hbm.at[cur], w1b.at[0], sems.at[0]).start()
        pltpu.make_async_copy(w2_hbm.at[cur], w2b.at[0], sems.at[0]).start()

    @pl.when(changed)
    def _():
        pltpu.make_async_copy(w0_hbm.at[cur], w0b.at[par], sems.at[par]).wait()
        pltpu.make_async_copy(w1_hbm.at[cur], w1b.at[par], sems.at[par]).wait()
        pltpu.make_async_copy(w2_hbm.at[cur], w2b.at[par], sems.at[par]).wait()

    @pl.when(jnp.logical_and(changed, nxe != cur))
    def _():
        nslot = 1 - par
        pltpu.make_async_copy(w0_hbm.at[nxe], w0b.at[nslot], sems.at[nslot]).start()
        pltpu.make_async_copy(w1_hbm.at[nxe], w1b.at[nslot], sems.at[nslot]).start()
        pltpu.make_async_copy(w2_hbm.at[nxe], w2b.at[nslot], sems.at[nslot]).start()

    v = x_ref[...]
    xl = lax.bitcast_convert_type((v & 0xFFFF).astype(jnp.uint16),
                                  jnp.bfloat16)
    xh = lax.bitcast_convert_type((v >> 16).astype(jnp.uint16), jnp.bfloat16)
    xb = jnp.concatenate([xl, xh], axis=1).astype(f32)

    h = jnp.dot(xb, w0b[par], preferred_element_type=f32)
    h = jnp.maximum(h + b0_ref[0], 0.0)
    h = jnp.dot(h, w1b[par], preferred_element_type=f32)
    h = jnp.maximum(h + b1_ref[0], 0.0)
    y_ref[...] = jnp.dot(h, w2b[par], preferred_element_type=f32) + b2_ref[0]


def _moe(te, nxe, par, xs, W0, b0, W1, b1, W2, b2):
    grid_spec = pltpu.PrefetchScalarGridSpec(
        num_scalar_prefetch=3,
        grid=(NT,),
        in_specs=[
            pl.BlockSpec((T, DP), lambda i, te, nxe, par: (i, 0)),
            pl.BlockSpec((1, 1, D), lambda i, te, nxe, par: (te[i], 0, 0)),
            pl.BlockSpec((1, 1, D), lambda i, te, nxe, par: (te[i], 0, 0)),
            pl.BlockSpec((1, 1, D), lambda i, te, nxe, par: (te[i], 0, 0)),
            pl.BlockSpec(memory_space=pltpu.MemorySpace.HBM),
            pl.BlockSpec(memory_space=pltpu.MemorySpace.HBM),
            pl.BlockSpec(memory_space=pltpu.MemorySpace.HBM),
        ],
        out_specs=pl.BlockSpec((T, D), lambda i, te, nxe, par: (i, 0)),
        scratch_shapes=[
            pltpu.VMEM((2, D, D), f32),
            pltpu.VMEM((2, D, D), f32),
            pltpu.VMEM((2, D, D), f32),
            pltpu.SemaphoreType.DMA((2,)),
        ],
    )
    return pl.pallas_call(
        _moe_body,
        grid_spec=grid_spec,
        out_shape=jax.ShapeDtypeStruct((PADDED_N, D), f32),
    )(te, nxe, par, xs, b0, b1, b2, W0, W1, W2)


def kernel(x, Wg, bg, W0, b0, W1, b1, W2, b2):
    slot, te2, nxe2, par2, xp = _route(x, jnp.swapaxes(Wg, 0, 1),
                                       bg.reshape(1, E))
    xs = _permute(xp, slot)
    ys = _moe(te2.reshape(LANES), nxe2.reshape(LANES), par2.reshape(LANES),
              xs, W0, b0.reshape(E, 1, D), W1, b1.reshape(E, 1, D),
              W2, b2.reshape(E, 1, D))
    return _unpermute(ys, slot)
